# baseline (speedup 1.0000x reference)
import jax
import jax.numpy as jnp
from jax.experimental import pallas as pl
from jax.experimental.pallas import tpu as pltpu

_LANE = 128
_POOL_BLOCK_BYTES = 8 * 1024 * 1024
_VMEM_CAP = 48 * 1024 * 1024


def _round_up(v, m):
    return -(-v // m) * m


def _pool(x, num_k, thw):
    N, C, HW = x.shape
    itemsize = jnp.dtype(x.dtype).itemsize
    out_shape = (jax.ShapeDtypeStruct((N, C, 1), jnp.float32),
                 jax.ShapeDtypeStruct((N, C, 1), jnp.float32))
    block_bytes = _round_up(C, 8) * _round_up(thw, _LANE) * itemsize
    vmem_limit = int(min(_VMEM_CAP, 2 * block_bytes + 8 * 1024 * 1024))
    cost = pl.CostEstimate(
        flops=2 * N * C * HW,
        transcendentals=0,
        bytes_accessed=N * C * HW * itemsize + 2 * N * C * 4,
    )

    if num_k == 1:
        ipb = 1
        for cand in (8, 4, 2):
            if N % cand == 0 and cand * C * HW * itemsize <= _POOL_BLOCK_BYTES:
                ipb = cand
                break
        vmem_limit = int(min(_VMEM_CAP,
                             2 * ipb * block_bytes + 8 * 1024 * 1024))

        if HW % (2 * _LANE) == 0:
            hh = HW // 2

            def body(a_ref, b_ref, s_ref, m_ref):
                av = a_ref[...]
                bv = b_ref[...]
                s_ref[...] = (jnp.sum(av, axis=2, keepdims=True)
                              + jnp.sum(bv, axis=2, keepdims=True))
                m_ref[...] = jnp.maximum(
                    jnp.max(av, axis=2, keepdims=True),
                    jnp.max(bv, axis=2, keepdims=True))

            return pl.pallas_call(
                body,
                out_shape=out_shape,
                grid=(N // ipb,),
                in_specs=[
                    pl.BlockSpec((ipb, C, hh), lambda n: (n, 0, 0)),
                    pl.BlockSpec((ipb, C, hh), lambda n: (n, 0, 1)),
                ],
                out_specs=(pl.BlockSpec((ipb, C, 1), lambda n: (n, 0, 0)),
                           pl.BlockSpec((ipb, C, 1), lambda n: (n, 0, 0))),
                compiler_params=pltpu.CompilerParams(
                    dimension_semantics=("arbitrary",),
                    vmem_limit_bytes=vmem_limit,
                ),
                cost_estimate=cost,
            )(x, x)

        def body(x_ref, s_ref, m_ref):
            xv = x_ref[...]
            s_ref[...] = jnp.sum(xv, axis=2, keepdims=True)
            m_ref[...] = jnp.max(xv, axis=2, keepdims=True)

        return pl.pallas_call(
            body,
            out_shape=out_shape,
            grid=(N // ipb,),
            in_specs=[pl.BlockSpec((ipb, C, HW), lambda n: (n, 0, 0))],
            out_specs=(pl.BlockSpec((ipb, C, 1), lambda n: (n, 0, 0)),
                       pl.BlockSpec((ipb, C, 1), lambda n: (n, 0, 0))),
            compiler_params=pltpu.CompilerParams(
                dimension_semantics=("arbitrary",),
                vmem_limit_bytes=vmem_limit,
            ),
            cost_estimate=cost,
        )(x)

    needs_mask = (HW % thw) != 0

    def body(x_ref, s_ref, m_ref, s_acc, m_acc):
        k = pl.program_id(1)

        @pl.when(k == 0)
        def _init():
            s_acc[...] = jnp.zeros_like(s_acc)
            m_acc[...] = jnp.full_like(m_acc, -jnp.inf)

        xv = x_ref[0]

        def _accum(xs, xm):
            s_acc[...] += jnp.sum(xs, axis=1, keepdims=True)
            m_acc[...] = jnp.maximum(m_acc[...],
                                     jnp.max(xm, axis=1, keepdims=True))

        if needs_mask:
            @pl.when(k < num_k - 1)
            def _full():
                _accum(xv.astype(jnp.float32), xv.astype(jnp.float32))

            @pl.when(k == num_k - 1)
            def _tail():
                lane = jax.lax.broadcasted_iota(jnp.int32, (C, thw), 1)
                valid = (k * thw + lane) < HW
                _accum(jnp.where(valid, xv.astype(jnp.float32), 0.0),
                       jnp.where(valid, xv.astype(jnp.float32), -jnp.inf))
        else:
            _accum(xv.astype(jnp.float32), xv.astype(jnp.float32))

        @pl.when(k == num_k - 1)
        def _fin():
            s_ref[0] = s_acc[...]
            m_ref[0] = m_acc[...]

    return pl.pallas_call(
        body,
        out_shape=out_shape,
        grid=(N, num_k),
        in_specs=[pl.BlockSpec((1, C, thw), lambda n, k: (n, 0, k))],
        out_specs=(pl.BlockSpec((1, C, 1), lambda n, k: (n, 0, 0)),
                   pl.BlockSpec((1, C, 1), lambda n, k: (n, 0, 0))),
        scratch_shapes=[pltpu.VMEM((C, 1), jnp.float32),
                        pltpu.VMEM((C, 1), jnp.float32)],
        compiler_params=pltpu.CompilerParams(
            dimension_semantics=("parallel", "arbitrary"),
            vmem_limit_bytes=vmem_limit,
        ),
        cost_estimate=cost,
    )(x)


def kernel(x_nchw, w1, w2):
    N, C, H, W = x_nchw.shape
    Cout = w2.shape[0]
    HW = H * W
    inv_hw = 1.0 / float(HW)
    itemsize = jnp.dtype(x_nchw.dtype).itemsize

    x = x_nchw.reshape(N, C, HW)

    c_pad = _round_up(C, 8 * max(1, 4 // itemsize))
    budget_lanes = max(
        _LANE, (_POOL_BLOCK_BYTES // (c_pad * itemsize)) // _LANE * _LANE)
    if budget_lanes >= HW:
        thw, num_k = HW, 1
    else:
        thw = budget_lanes
        num_k = int(pl.cdiv(HW, thw))

    psum, pmax = _pool(x, num_k, thw)

    sums = psum.reshape(N, C)
    maxs = pmax.reshape(N, C)

    def mlp_body(s_ref, m_ref, w1_ref, w2_ref, o_ref):
        avg = s_ref[...] * inv_hw
        p = jnp.concatenate([avg, m_ref[...]], axis=0)
        h = jax.lax.dot_general(
            p, w1_ref[...].astype(jnp.float32),
            (((1,), (1,)), ((), ())),
            preferred_element_type=jnp.float32)
        h = jnp.maximum(h, 0.0)
        o = jax.lax.dot_general(
            h, w2_ref[...].astype(jnp.float32),
            (((1,), (1,)), ((), ())),
            preferred_element_type=jnp.float32)
        gate = jax.nn.sigmoid(o[:N, :] + o[N:, :])
        o_ref[...] = gate.astype(o_ref.dtype)

    out = pl.pallas_call(
        mlp_body,
        out_shape=jax.ShapeDtypeStruct((N, Cout), x_nchw.dtype),
    )(sums, maxs, w1, w2)
    return out.reshape(N, Cout, 1, 1)


# speedup vs baseline: 1.2260x; 1.2260x over previous
import jax
import jax.numpy as jnp
from jax.experimental import pallas as pl
from jax.experimental.pallas import tpu as pltpu

_LANE = 128
_POOL_BLOCK_BYTES = 8 * 1024 * 1024
_VMEM_CAP = 48 * 1024 * 1024


def _round_up(v, m):
    return -(-v // m) * m


def _gate_rows(sum_rows, max_rows, w1, w2, inv_hw, n):
    p = jnp.concatenate([sum_rows * inv_hw, max_rows], axis=0)
    h = jax.lax.dot_general(p, w1, (((1,), (1,)), ((), ())),
                            preferred_element_type=jnp.float32)
    h = jnp.maximum(h, 0.0)
    o = jax.lax.dot_general(h, w2, (((1,), (1,)), ((), ())),
                            preferred_element_type=jnp.float32)
    return jax.nn.sigmoid(o[:n, :] + o[n:, :])


def kernel(x_nchw, w1, w2):
    N, C, H, W = x_nchw.shape
    Cout = w2.shape[0]
    HW = H * W
    inv_hw = 1.0 / float(HW)
    itemsize = jnp.dtype(x_nchw.dtype).itemsize

    x = x_nchw.reshape(N, C, HW)

    ipb = 1
    for cand in (8, 4, 2):
        if N % cand == 0 and cand * C * HW * itemsize <= _POOL_BLOCK_BYTES:
            ipb = cand
            break
    nsteps = N // ipb
    block_bytes = ipb * _round_up(C, 8) * _round_up(HW, _LANE) * itemsize
    vmem_limit = int(min(_VMEM_CAP, 2 * block_bytes + 8 * 1024 * 1024))

    if block_bytes <= _POOL_BLOCK_BYTES:
        def body2(x_ref, w1_ref, w2_ref, o_ref, s_rows, m_rows):
            k = pl.program_id(0)
            xv = x_ref[...]
            sl = pl.ds(k * ipb, ipb)
            s_rows[sl, :] = jnp.sum(xv, axis=2)
            m_rows[sl, :] = jnp.max(xv, axis=2)

            @pl.when(k == nsteps - 1)
            def _fin():
                gate = _gate_rows(s_rows[...], m_rows[...],
                                  w1_ref[...].astype(jnp.float32),
                                  w2_ref[...].astype(jnp.float32),
                                  inv_hw, N)
                o_ref[...] = gate.astype(o_ref.dtype)

        Cr = w1.shape[0]
        out = pl.pallas_call(
            body2,
            out_shape=jax.ShapeDtypeStruct((N, Cout), x_nchw.dtype),
            grid=(nsteps,),
            in_specs=[pl.BlockSpec((ipb, C, HW), lambda k: (k, 0, 0)),
                      pl.BlockSpec((Cr, C), lambda k: (0, 0)),
                      pl.BlockSpec((Cout, Cr), lambda k: (0, 0))],
            out_specs=pl.BlockSpec((N, Cout), lambda k: (0, 0)),
            scratch_shapes=[pltpu.VMEM((N, C), jnp.float32),
                            pltpu.VMEM((N, C), jnp.float32)],
            compiler_params=pltpu.CompilerParams(
                dimension_semantics=("arbitrary",),
                vmem_limit_bytes=vmem_limit,
            ),
            cost_estimate=pl.CostEstimate(
                flops=2 * N * C * HW + 4 * N * (C * Cr + Cr * Cout),
                transcendentals=N * Cout,
                bytes_accessed=N * C * HW * itemsize + N * Cout * itemsize,
            ),
        )(x, w1, w2)
        return out.reshape(N, Cout, 1, 1)

    thw = max(_LANE,
              (_POOL_BLOCK_BYTES // (_round_up(C, 8) * itemsize))
              // _LANE * _LANE)
    num_k = int(pl.cdiv(HW, thw))
    needs_mask = (HW % thw) != 0
    Cr = w1.shape[0]

    def pbody(x_ref, s_ref, m_ref, s_acc, m_acc):
        k = pl.program_id(1)

        @pl.when(k == 0)
        def _init():
            s_acc[...] = jnp.zeros_like(s_acc)
            m_acc[...] = jnp.full_like(m_acc, -jnp.inf)

        xv = x_ref[0]

        def _accum(xs, xm):
            s_acc[...] += jnp.sum(xs, axis=1, keepdims=True)
            m_acc[...] = jnp.maximum(m_acc[...],
                                     jnp.max(xm, axis=1, keepdims=True))

        if needs_mask:
            @pl.when(k < num_k - 1)
            def _full():
                _accum(xv.astype(jnp.float32), xv.astype(jnp.float32))

            @pl.when(k == num_k - 1)
            def _tail():
                lane = jax.lax.broadcasted_iota(jnp.int32, (C, thw), 1)
                valid = (k * thw + lane) < HW
                _accum(jnp.where(valid, xv.astype(jnp.float32), 0.0),
                       jnp.where(valid, xv.astype(jnp.float32), -jnp.inf))
        else:
            _accum(xv.astype(jnp.float32), xv.astype(jnp.float32))

        @pl.when(k == num_k - 1)
        def _fin():
            s_ref[0] = s_acc[...]
            m_ref[0] = m_acc[...]

    psum, pmax = pl.pallas_call(
        pbody,
        out_shape=(jax.ShapeDtypeStruct((N, C, 1), jnp.float32),
                   jax.ShapeDtypeStruct((N, C, 1), jnp.float32)),
        grid=(N, num_k),
        in_specs=[pl.BlockSpec((1, C, thw), lambda n, k: (n, 0, k))],
        out_specs=(pl.BlockSpec((1, C, 1), lambda n, k: (n, 0, 0)),
                   pl.BlockSpec((1, C, 1), lambda n, k: (n, 0, 0))),
        scratch_shapes=[pltpu.VMEM((C, 1), jnp.float32),
                        pltpu.VMEM((C, 1), jnp.float32)],
        compiler_params=pltpu.CompilerParams(
            dimension_semantics=("parallel", "arbitrary"),
            vmem_limit_bytes=int(min(
                _VMEM_CAP,
                2 * _round_up(C, 8) * thw * itemsize + 8 * 1024 * 1024)),
        ),
    )(x)

    sums = psum.reshape(N, C)
    maxs = pmax.reshape(N, C)

    def mlp_body(s_ref, m_ref, w1_ref, w2_ref, o_ref):
        gate = _gate_rows(s_ref[...], m_ref[...],
                          w1_ref[...].astype(jnp.float32),
                          w2_ref[...].astype(jnp.float32), inv_hw, N)
        o_ref[...] = gate.astype(o_ref.dtype)

    out = pl.pallas_call(
        mlp_body,
        out_shape=jax.ShapeDtypeStruct((N, Cout), x_nchw.dtype),
    )(sums, maxs, w1, w2)
    return out.reshape(N, Cout, 1, 1)
